# baseline (speedup 1.0000x reference)
import functools

import jax
import jax.numpy as jnp
from jax import lax
from jax.experimental import pallas as pl
from jax.experimental.pallas import tpu as pltpu
from jax.experimental.pallas import tpu_sc as plsc

EMBED_DIM = 64
NUM_CORES = 2
NUM_SUBCORES = 16
NUM_WORKERS = NUM_CORES * NUM_SUBCORES
CHUNK = 128


@functools.partial(jax.jit, static_argnames=())
def _sc_gather(table, idx_flat):
    total = idx_flat.shape[0]
    assert total % (8 * NUM_WORKERS) == 0
    b_per_w = total // NUM_WORKERS
    n_chunks = b_per_w // CHUNK
    assert n_chunks * CHUNK == b_per_w

    mesh = plsc.VectorSubcoreMesh(
        core_axis_name="c", subcore_axis_name="s",
        num_cores=NUM_CORES, num_subcores=NUM_SUBCORES)

    @functools.partial(
        pl.kernel,
        out_type=jax.ShapeDtypeStruct((total, EMBED_DIM), jnp.float32),
        mesh=mesh,
        compiler_params=pltpu.CompilerParams(use_tc_tiling_on_sc=False),
        scratch_types=[
            pltpu.VMEM((b_per_w,), jnp.int32),
            pltpu.VMEM((CHUNK, EMBED_DIM), jnp.float32),
            pltpu.SemaphoreType.DMA,
        ],
    )
    def gather_kernel(table_hbm, idx_hbm, out_hbm, idx_v, rows_v, gsem):
        wid = lax.axis_index("s") * NUM_CORES + lax.axis_index("c")
        base = wid * b_per_w
        pltpu.sync_copy(idx_hbm.at[pl.ds(base, b_per_w)], idx_v)

        @pl.loop(0, n_chunks)
        def _chunk(c):
            off = c * CHUNK
            idx_c = idx_v.at[pl.ds(off, CHUNK)]
            pltpu.async_copy(table_hbm.at[idx_c], rows_v, gsem).wait()
            pltpu.sync_copy(rows_v, out_hbm.at[pl.ds(base + off, CHUNK)])

    return gather_kernel(table, idx_flat)


def kernel(indices, vectors):
    batch, hist = indices.shape
    idx_flat = indices.reshape(-1).astype(jnp.int32)
    out = _sc_gather(vectors, idx_flat)
    return out.reshape(batch, hist, EMBED_DIM)


# speedup vs baseline: 1.0499x; 1.0499x over previous
import functools

import jax
import jax.numpy as jnp
from jax import lax
from jax.experimental import pallas as pl
from jax.experimental.pallas import tpu as pltpu
from jax.experimental.pallas import tpu_sc as plsc

EMBED_DIM = 64
NUM_CORES = 2
NUM_SUBCORES = 16
NUM_WORKERS = NUM_CORES * NUM_SUBCORES
CHUNK = 128


@functools.partial(jax.jit, static_argnames=())
def _sc_gather(table, idx_flat):
    total = idx_flat.shape[0]
    assert total % (8 * NUM_WORKERS) == 0
    b_per_w = total // NUM_WORKERS
    n_chunks = b_per_w // CHUNK
    assert n_chunks * CHUNK == b_per_w

    mesh = plsc.VectorSubcoreMesh(
        core_axis_name="c", subcore_axis_name="s",
        num_cores=NUM_CORES, num_subcores=NUM_SUBCORES)

    nbuf = 5
    assert n_chunks % nbuf == 0 and n_chunks >= 2 * nbuf

    @functools.partial(
        pl.kernel,
        out_type=jax.ShapeDtypeStruct((total, EMBED_DIM), jnp.float32),
        mesh=mesh,
        compiler_params=pltpu.CompilerParams(use_tc_tiling_on_sc=False),
        scratch_types=[
            pltpu.VMEM((b_per_w,), jnp.int32),
            pltpu.VMEM((nbuf, CHUNK, EMBED_DIM), jnp.float32),
            pltpu.SemaphoreType.DMA,
            pltpu.SemaphoreType.DMA,
        ],
    )
    def gather_kernel(table_hbm, idx_hbm, out_hbm, idx_v, rows_v, gsem, wsem):
        wid = lax.axis_index("s") * NUM_CORES + lax.axis_index("c")
        base = wid * b_per_w
        pltpu.sync_copy(idx_hbm.at[pl.ds(base, b_per_w)], idx_v)

        def start_gather(chunk, buf):
            idx_c = idx_v.at[pl.ds(chunk * CHUNK, CHUNK)]
            pltpu.async_copy(table_hbm.at[idx_c], rows_v.at[buf], gsem)

        for b in range(nbuf):
            start_gather(b, b)

        @pl.loop(0, n_chunks, step=nbuf)
        def _chunk(c):
            for b in range(nbuf):
                chunk = c + b
                off = chunk * CHUNK
                out_slice = out_hbm.at[pl.ds(base + off, CHUNK)]
                pltpu.make_async_copy(
                    table_hbm.at[idx_v.at[pl.ds(off, CHUNK)]],
                    rows_v.at[b], gsem).wait()
                pltpu.async_copy(rows_v.at[b], out_slice, wsem)
                nxt = chunk + nbuf

                @pl.when(nxt < n_chunks)
                def _refill():
                    pltpu.make_async_copy(rows_v.at[b], out_slice, wsem).wait()
                    start_gather(nxt, b)

        for b in range(nbuf):
            pltpu.make_async_copy(
                rows_v.at[b],
                out_hbm.at[pl.ds(base, CHUNK)], wsem).wait()

    return gather_kernel(table, idx_flat)


def kernel(indices, vectors):
    batch, hist = indices.shape
    idx_flat = indices.reshape(-1).astype(jnp.int32)
    out = _sc_gather(vectors, idx_flat)
    return out.reshape(batch, hist, EMBED_DIM)
